# Initial kernel scaffold; baseline (speedup 1.0000x reference)
import jax
import jax.numpy as jnp
from jax.experimental import pallas as pl


def kernel(x, edge_index, W_in, b_in, Wc, bc):
    raise NotImplementedError("write your pallas kernel here")


# speedup vs baseline: 17.2492x; 17.2492x over previous
import functools

import jax
import jax.numpy as jnp
from jax import lax
from jax.experimental import pallas as pl
from jax.experimental.pallas import tpu as pltpu
from jax.experimental.pallas import tpu_sc as plsc

N = 10000
E = 320000
F_IN = 128
D = 64
L = 4

NC = 2
NS = 16
NW = NC * NS
EPW = E // NW
CH = 80
NCH = EPW // CH
NPAD = 10240
RPT = NPAD // NS

_MESH = plsc.VectorSubcoreMesh(core_axis_name="c", subcore_axis_name="s")


@functools.partial(
    pl.kernel,
    mesh=_MESH,
    out_type=jax.ShapeDtypeStruct((NC, NPAD, 16), jnp.float32),
    scratch_types=[
        pltpu.VMEM((NCH, CH), jnp.int32),
        pltpu.VMEM((CH, 16), jnp.float32),
        pltpu.VMEM((RPT, 16), jnp.float32),
        pltpu.VMEM_SHARED((NPAD, 16), jnp.float32),
        pltpu.SemaphoreType.DMA,
    ],
    compiler_params=pltpu.CompilerParams(use_tc_tiling_on_sc=False),
)
def _deg_call(dst_hbm, zeros_hbm, ones_hbm, out_hbm,
              idx_v, ones_v, buf_v, acc_sh, sem):
    c = lax.axis_index("c")
    s = lax.axis_index("s")
    wid = s * NC + c
    rows = pl.ds(s * RPT, RPT)

    pltpu.sync_copy(dst_hbm.at[wid], idx_v)
    pltpu.sync_copy(ones_hbm, ones_v)
    pltpu.sync_copy(zeros_hbm.at[rows], buf_v)
    pltpu.sync_copy(buf_v, acc_sh.at[rows])
    plsc.subcore_barrier()

    def body(j, carry):
        pltpu.sync_copy(ones_v, acc_sh.at[idx_v.at[j]], add=True)
        return carry

    lax.fori_loop(0, NCH, body, 0)
    plsc.subcore_barrier()
    pltpu.sync_copy(acc_sh.at[rows], buf_v)
    pltpu.sync_copy(buf_v, out_hbm.at[c].at[rows])


@functools.partial(
    pl.kernel,
    mesh=_MESH,
    out_type=jax.ShapeDtypeStruct((NC, NPAD, D), jnp.float32),
    scratch_types=[
        pltpu.VMEM((NCH, CH), jnp.int32),
        pltpu.VMEM((NCH, CH), jnp.int32),
        pltpu.VMEM((CH, D), jnp.float32),
        pltpu.VMEM((RPT, D), jnp.float32),
        pltpu.VMEM_SHARED((NPAD, D), jnp.float32),
        pltpu.SemaphoreType.DMA,
    ],
    compiler_params=pltpu.CompilerParams(use_tc_tiling_on_sc=False),
)
def _edge_call(src_hbm, dst_hbm, mh_hbm, zeros_hbm, out_hbm,
               si_v, di_v, rows_v, buf_v, acc_sh, sem):
    c = lax.axis_index("c")
    s = lax.axis_index("s")
    wid = s * NC + c
    rows = pl.ds(s * RPT, RPT)

    pltpu.sync_copy(src_hbm.at[wid], si_v)
    pltpu.sync_copy(dst_hbm.at[wid], di_v)
    pltpu.sync_copy(zeros_hbm.at[rows], buf_v)
    pltpu.sync_copy(buf_v, acc_sh.at[rows])
    plsc.subcore_barrier()

    def body(j, carry):
        pltpu.async_copy(mh_hbm.at[si_v.at[j]], rows_v, sem).wait()
        pltpu.sync_copy(rows_v, acc_sh.at[di_v.at[j]], add=True)
        return carry

    lax.fori_loop(0, NCH, body, 0)
    plsc.subcore_barrier()
    pltpu.sync_copy(acc_sh.at[rows], buf_v)
    pltpu.sync_copy(buf_v, out_hbm.at[c].at[rows])


_BR = 1000
_GRID = N // _BR


def _tc_in_body(x_ref, w_ref, b_ref, o_ref):
    h = jnp.dot(x_ref[...], w_ref[...],
                preferred_element_type=jnp.float32) + b_ref[...]
    o_ref[...] = jnp.where(h >= 0, h, 0.01 * h)


def _tc_in(x, w_in, b_in):
    return pl.pallas_call(
        _tc_in_body,
        grid=(_GRID,),
        in_specs=[
            pl.BlockSpec((_BR, F_IN), lambda i: (i, 0)),
            pl.BlockSpec((F_IN, D), lambda i: (0, 0)),
            pl.BlockSpec((1, D), lambda i: (0, 0)),
        ],
        out_specs=pl.BlockSpec((_BR, D), lambda i: (i, 0)),
        out_shape=jax.ShapeDtypeStruct((N, D), jnp.float32),
    )(x, w_in, b_in)


def _tc_prep_body(h_ref, w_ref, d0_ref, d1_ref, mh_ref, dis_ref):
    deg = 1.0 + d0_ref[...] + d1_ref[...]
    dis = lax.rsqrt(deg)
    m = jnp.dot(h_ref[...], w_ref[...], preferred_element_type=jnp.float32)
    mh_ref[...] = dis * m
    dis_ref[...] = dis


def _tc_prep(h, w, d0, d1):
    return pl.pallas_call(
        _tc_prep_body,
        grid=(_GRID,),
        in_specs=[
            pl.BlockSpec((_BR, D), lambda i: (i, 0)),
            pl.BlockSpec((D, D), lambda i: (0, 0)),
            pl.BlockSpec((_BR, 1), lambda i: (i, 0)),
            pl.BlockSpec((_BR, 1), lambda i: (i, 0)),
        ],
        out_specs=[
            pl.BlockSpec((_BR, D), lambda i: (i, 0)),
            pl.BlockSpec((_BR, 1), lambda i: (i, 0)),
        ],
        out_shape=[
            jax.ShapeDtypeStruct((N, D), jnp.float32),
            jax.ShapeDtypeStruct((N, 1), jnp.float32),
        ],
    )(h, w, d0, d1)


def _tc_layer_body(a_ref, mh_ref, dis_ref, b_ref, w_ref, h_ref, mhn_ref):
    dis = dis_ref[...]
    pre = (a_ref[0] + a_ref[1] + mh_ref[...]) * dis + b_ref[...]
    h = jnp.where(pre >= 0, pre, 0.01 * pre)
    h_ref[...] = h
    m = jnp.dot(h, w_ref[...], preferred_element_type=jnp.float32)
    mhn_ref[...] = dis * m


def _tc_layer(accp, mh, dis, b, w):
    return pl.pallas_call(
        _tc_layer_body,
        grid=(_GRID,),
        in_specs=[
            pl.BlockSpec((NC, _BR, D), lambda i: (0, i, 0)),
            pl.BlockSpec((_BR, D), lambda i: (i, 0)),
            pl.BlockSpec((_BR, 1), lambda i: (i, 0)),
            pl.BlockSpec((1, D), lambda i: (0, 0)),
            pl.BlockSpec((D, D), lambda i: (0, 0)),
        ],
        out_specs=[
            pl.BlockSpec((_BR, D), lambda i: (i, 0)),
            pl.BlockSpec((_BR, D), lambda i: (i, 0)),
        ],
        out_shape=[
            jax.ShapeDtypeStruct((N, D), jnp.float32),
            jax.ShapeDtypeStruct((N, D), jnp.float32),
        ],
    )(accp, mh, dis, b, w)


def kernel(x, edge_index, W_in, b_in, Wc, bc):
    src = edge_index[0].reshape(NW, NCH, CH)
    dst = edge_index[1].reshape(NW, NCH, CH)
    zeros_d = jnp.zeros((NPAD, D), jnp.float32)
    zeros_16 = jnp.zeros((NPAD, 16), jnp.float32)
    ones_16 = jnp.ones((CH, 16), jnp.float32)

    degp = _deg_call(dst, zeros_16, ones_16)
    d0 = degp[0, :N, 0:1]
    d1 = degp[1, :N, 0:1]

    h0 = _tc_in(x, W_in, b_in.reshape(1, D))
    mh, dis = _tc_prep(h0, Wc[0], d0, d1)

    hs = []
    for k in range(L):
        accp = _edge_call(src, dst, mh, zeros_d)[:, :N]
        h, mh = _tc_layer(accp, mh, dis, bc[k].reshape(1, D),
                          Wc[(k + 1) % L])
        hs.append(h)

    xs = jnp.stack([h.reshape(-1) for h in hs], axis=1)
    return (hs[-1], hs[-1], xs)


# speedup vs baseline: 23.6395x; 1.3705x over previous
import functools

import jax
import jax.numpy as jnp
from jax import lax
from jax.experimental import pallas as pl
from jax.experimental.pallas import tpu as pltpu
from jax.experimental.pallas import tpu_sc as plsc

N = 10000
E = 320000
F_IN = 128
D = 64
L = 4

NC = 2
NS = 16
NW = NC * NS
EPW = E // NW
CH = 125
NCH = EPW // CH
NPAD = 10240
RPT = NPAD // NS

_MESH = plsc.VectorSubcoreMesh(core_axis_name="c", subcore_axis_name="s")


@functools.partial(
    pl.kernel,
    mesh=_MESH,
    out_type=jax.ShapeDtypeStruct((NC, NPAD, 16), jnp.float32),
    scratch_types=[
        pltpu.VMEM((NCH, CH), jnp.int32),
        pltpu.VMEM((CH, 16), jnp.float32),
        pltpu.VMEM((RPT, 16), jnp.float32),
        pltpu.VMEM_SHARED((NPAD, 16), jnp.float32),
        pltpu.SemaphoreType.DMA,
    ],
    compiler_params=pltpu.CompilerParams(use_tc_tiling_on_sc=False),
)
def _deg_call(dst_hbm, zeros_hbm, ones_hbm, out_hbm,
              idx_v, ones_v, buf_v, acc_sh, sem):
    c = lax.axis_index("c")
    s = lax.axis_index("s")
    wid = s * NC + c
    rows = pl.ds(s * RPT, RPT)

    pltpu.sync_copy(dst_hbm.at[wid], idx_v)
    pltpu.sync_copy(ones_hbm, ones_v)
    pltpu.sync_copy(zeros_hbm.at[rows], buf_v)
    pltpu.sync_copy(buf_v, acc_sh.at[rows])
    plsc.subcore_barrier()

    def body(j, carry):
        pltpu.sync_copy(ones_v, acc_sh.at[idx_v.at[j]], add=True)
        return carry

    lax.fori_loop(0, NCH, body, 0)
    plsc.subcore_barrier()
    pltpu.sync_copy(acc_sh.at[rows], buf_v)
    pltpu.sync_copy(buf_v, out_hbm.at[c].at[rows])


@functools.partial(
    pl.kernel,
    mesh=_MESH,
    out_type=jax.ShapeDtypeStruct((NC, NPAD, D), jnp.float32),
    scratch_types=[
        pltpu.VMEM((NCH, CH), jnp.int32),
        pltpu.VMEM((NCH, CH), jnp.int32),
        pltpu.VMEM((CH, D), jnp.float32),
        pltpu.VMEM((CH, D), jnp.float32),
        pltpu.VMEM((RPT, D), jnp.float32),
        pltpu.VMEM_SHARED((NPAD, D), jnp.float32),
        pltpu.SemaphoreType.DMA,
        pltpu.SemaphoreType.DMA,
    ],
    compiler_params=pltpu.CompilerParams(use_tc_tiling_on_sc=False),
)
def _edge_call(src_hbm, dst_hbm, mh_hbm, zeros_hbm, out_hbm,
               si_v, di_v, r0, r1, buf_v, acc_sh, sem0, sem1):
    c = lax.axis_index("c")
    s = lax.axis_index("s")
    wid = s * NC + c
    rows = pl.ds(s * RPT, RPT)

    pltpu.sync_copy(src_hbm.at[wid], si_v)
    pltpu.sync_copy(dst_hbm.at[wid], di_v)
    pltpu.sync_copy(zeros_hbm.at[rows], buf_v)
    pltpu.sync_copy(buf_v, acc_sh.at[rows])
    plsc.subcore_barrier()

    pltpu.async_copy(mh_hbm.at[si_v.at[0]], r0, sem0)

    def body(i, carry):
        j0 = 2 * i
        j1 = j0 + 1
        pltpu.make_async_copy(mh_hbm.at[si_v.at[j0]], r0, sem0).wait()
        pltpu.async_copy(mh_hbm.at[si_v.at[j1]], r1, sem1)
        pltpu.sync_copy(r0, acc_sh.at[di_v.at[j0]], add=True)
        pltpu.make_async_copy(mh_hbm.at[si_v.at[j1]], r1, sem1).wait()

        @pl.when(j0 + 2 < NCH)
        def _():
            pltpu.async_copy(mh_hbm.at[si_v.at[j0 + 2]], r0, sem0)

        pltpu.sync_copy(r1, acc_sh.at[di_v.at[j1]], add=True)
        return carry

    lax.fori_loop(0, NCH // 2, body, 0)
    plsc.subcore_barrier()
    pltpu.sync_copy(acc_sh.at[rows], buf_v)
    pltpu.sync_copy(buf_v, out_hbm.at[c].at[rows])


_BR = 1000
_GRID = N // _BR


def _tc_in_body(x_ref, w_ref, b_ref, o_ref):
    h = jnp.dot(x_ref[...], w_ref[...],
                preferred_element_type=jnp.float32) + b_ref[...]
    o_ref[...] = jnp.where(h >= 0, h, 0.01 * h)


def _tc_in(x, w_in, b_in):
    return pl.pallas_call(
        _tc_in_body,
        grid=(_GRID,),
        in_specs=[
            pl.BlockSpec((_BR, F_IN), lambda i: (i, 0)),
            pl.BlockSpec((F_IN, D), lambda i: (0, 0)),
            pl.BlockSpec((1, D), lambda i: (0, 0)),
        ],
        out_specs=pl.BlockSpec((_BR, D), lambda i: (i, 0)),
        out_shape=jax.ShapeDtypeStruct((N, D), jnp.float32),
    )(x, w_in, b_in)


def _tc_prep_body(h_ref, w_ref, d0_ref, d1_ref, mh_ref, dis_ref):
    deg = 1.0 + d0_ref[...] + d1_ref[...]
    dis = lax.rsqrt(deg)
    m = jnp.dot(h_ref[...], w_ref[...], preferred_element_type=jnp.float32)
    mh_ref[...] = dis * m
    dis_ref[...] = dis


def _tc_prep(h, w, d0, d1):
    return pl.pallas_call(
        _tc_prep_body,
        grid=(_GRID,),
        in_specs=[
            pl.BlockSpec((_BR, D), lambda i: (i, 0)),
            pl.BlockSpec((D, D), lambda i: (0, 0)),
            pl.BlockSpec((_BR, 1), lambda i: (i, 0)),
            pl.BlockSpec((_BR, 1), lambda i: (i, 0)),
        ],
        out_specs=[
            pl.BlockSpec((_BR, D), lambda i: (i, 0)),
            pl.BlockSpec((_BR, 1), lambda i: (i, 0)),
        ],
        out_shape=[
            jax.ShapeDtypeStruct((N, D), jnp.float32),
            jax.ShapeDtypeStruct((N, 1), jnp.float32),
        ],
    )(h, w, d0, d1)


def _tc_layer_body(a_ref, mh_ref, dis_ref, b_ref, w_ref, h_ref, mhn_ref):
    dis = dis_ref[...]
    pre = (a_ref[0] + a_ref[1] + mh_ref[...]) * dis + b_ref[...]
    h = jnp.where(pre >= 0, pre, 0.01 * pre)
    h_ref[...] = h
    m = jnp.dot(h, w_ref[...], preferred_element_type=jnp.float32)
    mhn_ref[...] = dis * m


def _tc_layer(accp, mh, dis, b, w):
    return pl.pallas_call(
        _tc_layer_body,
        grid=(_GRID,),
        in_specs=[
            pl.BlockSpec((NC, _BR, D), lambda i: (0, i, 0)),
            pl.BlockSpec((_BR, D), lambda i: (i, 0)),
            pl.BlockSpec((_BR, 1), lambda i: (i, 0)),
            pl.BlockSpec((1, D), lambda i: (0, 0)),
            pl.BlockSpec((D, D), lambda i: (0, 0)),
        ],
        out_specs=[
            pl.BlockSpec((_BR, D), lambda i: (i, 0)),
            pl.BlockSpec((_BR, D), lambda i: (i, 0)),
        ],
        out_shape=[
            jax.ShapeDtypeStruct((N, D), jnp.float32),
            jax.ShapeDtypeStruct((N, D), jnp.float32),
        ],
    )(accp, mh, dis, b, w)


def kernel(x, edge_index, W_in, b_in, Wc, bc):
    src = edge_index[0].reshape(NW, NCH, CH)
    dst = edge_index[1].reshape(NW, NCH, CH)
    zeros_d = jnp.zeros((NPAD, D), jnp.float32)
    zeros_16 = jnp.zeros((NPAD, 16), jnp.float32)
    ones_16 = jnp.ones((CH, 16), jnp.float32)

    degp = _deg_call(dst, zeros_16, ones_16)
    d0 = degp[0, :N, 0:1]
    d1 = degp[1, :N, 0:1]

    h0 = _tc_in(x, W_in, b_in.reshape(1, D))
    mh, dis = _tc_prep(h0, Wc[0], d0, d1)

    hs = []
    for k in range(L):
        accp = _edge_call(src, dst, mh, zeros_d)[:, :N]
        h, mh = _tc_layer(accp, mh, dis, bc[k].reshape(1, D),
                          Wc[(k + 1) % L])
        hs.append(h)

    xs = jnp.stack([h.reshape(-1) for h in hs], axis=1)
    return (hs[-1], hs[-1], xs)


# speedup vs baseline: 24.4011x; 1.0322x over previous
import functools

import jax
import jax.numpy as jnp
from jax import lax
from jax.experimental import pallas as pl
from jax.experimental.pallas import tpu as pltpu
from jax.experimental.pallas import tpu_sc as plsc

N = 10000
E = 320000
F_IN = 128
D = 64
L = 4

NC = 2
NS = 16
NW = NC * NS
EPW = E // NW
CH = 125
NCH = EPW // CH
NPAD = 10240
NB = 2
RPT = NPAD // NS

_MESH = plsc.VectorSubcoreMesh(core_axis_name="c", subcore_axis_name="s")


@functools.partial(
    pl.kernel,
    mesh=_MESH,
    out_type=jax.ShapeDtypeStruct((NC, NPAD, 16), jnp.float32),
    scratch_types=[
        pltpu.VMEM((NCH, CH), jnp.int32),
        pltpu.VMEM((CH, 16), jnp.float32),
        pltpu.VMEM((RPT, 16), jnp.float32),
        pltpu.VMEM_SHARED((NPAD, 16), jnp.float32),
        pltpu.SemaphoreType.DMA,
    ],
    compiler_params=pltpu.CompilerParams(use_tc_tiling_on_sc=False),
)
def _deg_call(dst_hbm, zeros_hbm, ones_hbm, out_hbm,
              idx_v, ones_v, buf_v, acc_sh, sem):
    c = lax.axis_index("c")
    s = lax.axis_index("s")
    wid = s * NC + c
    rows = pl.ds(s * RPT, RPT)

    pltpu.sync_copy(dst_hbm.at[wid], idx_v)
    pltpu.sync_copy(ones_hbm, ones_v)
    pltpu.sync_copy(zeros_hbm.at[rows], buf_v)
    pltpu.sync_copy(buf_v, acc_sh.at[rows])
    plsc.subcore_barrier()

    def body(j, carry):
        pltpu.sync_copy(ones_v, acc_sh.at[idx_v.at[j]], add=True)
        return carry

    lax.fori_loop(0, NCH, body, 0)
    plsc.subcore_barrier()
    pltpu.sync_copy(acc_sh.at[rows], buf_v)
    pltpu.sync_copy(buf_v, out_hbm.at[c].at[rows])


@functools.partial(
    pl.kernel,
    mesh=_MESH,
    out_type=jax.ShapeDtypeStruct((NC, NPAD, D), jnp.float32),
    scratch_types=[
        pltpu.VMEM((NCH, CH), jnp.int32),
        pltpu.VMEM((NCH, CH), jnp.int32),
        [pltpu.VMEM((CH, D), jnp.float32)] * NB,
        pltpu.VMEM((RPT, D), jnp.float32),
        pltpu.VMEM_SHARED((NPAD, D), jnp.float32),
        [pltpu.SemaphoreType.DMA] * NB,
        [pltpu.SemaphoreType.DMA] * NB,
    ],
    compiler_params=pltpu.CompilerParams(use_tc_tiling_on_sc=False),
)
def _edge_call(src_hbm, dst_hbm, mh_hbm, zeros_hbm, out_hbm,
               si_v, di_v, rb, buf_v, acc_sh, gsem, ssem):
    c = lax.axis_index("c")
    s = lax.axis_index("s")
    wid = s * NC + c
    rows = pl.ds(s * RPT, RPT)

    pltpu.sync_copy(src_hbm.at[wid], si_v)
    pltpu.sync_copy(dst_hbm.at[wid], di_v)
    pltpu.sync_copy(zeros_hbm.at[rows], buf_v)
    pltpu.sync_copy(buf_v, acc_sh.at[rows])
    plsc.subcore_barrier()

    for b in range(NB):
        pltpu.async_copy(mh_hbm.at[si_v.at[b]], rb[b], gsem[b])

    def body(i, carry):
        base = i * NB
        for b in range(NB):
            j = base + b
            pltpu.make_async_copy(mh_hbm.at[si_v.at[j]], rb[b],
                                  gsem[b]).wait()
            pltpu.async_copy(rb[b], acc_sh.at[di_v.at[j]], ssem[b],
                             add=True)
        for b in range(NB):
            nxt = base + NB + b
            pltpu.make_async_copy(rb[b], acc_sh.at[di_v.at[base + b]],
                                  ssem[b]).wait()

            @pl.when(nxt < NCH)
            def _():
                pltpu.async_copy(mh_hbm.at[si_v.at[nxt]], rb[b], gsem[b])

        return carry

    lax.fori_loop(0, NCH // NB, body, 0)
    plsc.subcore_barrier()
    pltpu.sync_copy(acc_sh.at[rows], buf_v)
    pltpu.sync_copy(buf_v, out_hbm.at[c].at[rows])


_BR = 1000
_GRID = N // _BR


def _tc_front_body(x_ref, wi_ref, bi_ref, w0_ref, d0_ref, d1_ref,
                   mh_ref, dis_ref):
    pre = jnp.dot(x_ref[...], wi_ref[...],
                  preferred_element_type=jnp.float32) + bi_ref[...]
    h = jnp.where(pre >= 0, pre, 0.01 * pre)
    deg = 1.0 + d0_ref[...] + d1_ref[...]
    dis = lax.rsqrt(deg)
    m = jnp.dot(h, w0_ref[...], preferred_element_type=jnp.float32)
    mh_ref[...] = dis * m
    dis_ref[...] = dis


def _tc_front(x, w_in, b_in, w0, d0, d1):
    return pl.pallas_call(
        _tc_front_body,
        grid=(_GRID,),
        in_specs=[
            pl.BlockSpec((_BR, F_IN), lambda i: (i, 0)),
            pl.BlockSpec((F_IN, D), lambda i: (0, 0)),
            pl.BlockSpec((1, D), lambda i: (0, 0)),
            pl.BlockSpec((D, D), lambda i: (0, 0)),
            pl.BlockSpec((_BR, 1), lambda i: (i, 0)),
            pl.BlockSpec((_BR, 1), lambda i: (i, 0)),
        ],
        out_specs=[
            pl.BlockSpec((_BR, D), lambda i: (i, 0)),
            pl.BlockSpec((_BR, 1), lambda i: (i, 0)),
        ],
        out_shape=[
            jax.ShapeDtypeStruct((N, D), jnp.float32),
            jax.ShapeDtypeStruct((N, 1), jnp.float32),
        ],
    )(x, w_in, b_in, w0, d0, d1)


def _tc_layer_body(a_ref, mh_ref, dis_ref, b_ref, w_ref, h_ref, mhn_ref):
    dis = dis_ref[...]
    pre = (a_ref[0] + a_ref[1] + mh_ref[...]) * dis + b_ref[...]
    h = jnp.where(pre >= 0, pre, 0.01 * pre)
    h_ref[...] = h
    m = jnp.dot(h, w_ref[...], preferred_element_type=jnp.float32)
    mhn_ref[...] = dis * m


def _tc_layer(accp, mh, dis, b, w):
    return pl.pallas_call(
        _tc_layer_body,
        grid=(_GRID,),
        in_specs=[
            pl.BlockSpec((NC, _BR, D), lambda i: (0, i, 0)),
            pl.BlockSpec((_BR, D), lambda i: (i, 0)),
            pl.BlockSpec((_BR, 1), lambda i: (i, 0)),
            pl.BlockSpec((1, D), lambda i: (0, 0)),
            pl.BlockSpec((D, D), lambda i: (0, 0)),
        ],
        out_specs=[
            pl.BlockSpec((_BR, D), lambda i: (i, 0)),
            pl.BlockSpec((_BR, D), lambda i: (i, 0)),
        ],
        out_shape=[
            jax.ShapeDtypeStruct((N, D), jnp.float32),
            jax.ShapeDtypeStruct((N, D), jnp.float32),
        ],
    )(accp, mh, dis, b, w)


def kernel(x, edge_index, W_in, b_in, Wc, bc):
    src = edge_index[0].reshape(NW, NCH, CH)
    dst = edge_index[1].reshape(NW, NCH, CH)
    zeros_d = jnp.zeros((NPAD, D), jnp.float32)
    zeros_16 = jnp.zeros((NPAD, 16), jnp.float32)
    ones_16 = jnp.ones((CH, 16), jnp.float32)

    degp = _deg_call(dst, zeros_16, ones_16)
    d0 = degp[0, :N, 0:1]
    d1 = degp[1, :N, 0:1]
    mh, dis = _tc_front(x, W_in, b_in.reshape(1, D), Wc[0], d0, d1)

    hs = []
    for k in range(L):
        accp = _edge_call(src, dst, mh, zeros_d)[:, :N]
        h, mh = _tc_layer(accp, mh, dis, bc[k].reshape(1, D),
                          Wc[(k + 1) % L])
        hs.append(h)

    xs = jnp.stack([h.reshape(-1) for h in hs], axis=1)
    return (hs[-1], hs[-1], xs)


# speedup vs baseline: 27.6868x; 1.1347x over previous
import functools

import jax
import jax.numpy as jnp
from jax import lax
from jax.experimental import pallas as pl
from jax.experimental.pallas import tpu as pltpu
from jax.experimental.pallas import tpu_sc as plsc

N = 10000
E = 320000
F_IN = 128
D = 64
L = 4

NC = 2
NS = 16
NW = NC * NS
EPW = E // NW
CH = 125
NCH = EPW // CH
NPAD = 10240
NB = 3
RPT = NPAD // NS

_MESH = plsc.VectorSubcoreMesh(core_axis_name="c", subcore_axis_name="s")


@functools.partial(
    pl.kernel,
    mesh=_MESH,
    out_type=jax.ShapeDtypeStruct((NC, NPAD, 16), jnp.float32),
    scratch_types=[
        pltpu.VMEM((NCH, CH), jnp.int32),
        pltpu.VMEM((CH, 16), jnp.float32),
        pltpu.VMEM((RPT, 16), jnp.float32),
        pltpu.VMEM_SHARED((NPAD, 16), jnp.float32),
        pltpu.SemaphoreType.DMA,
    ],
    compiler_params=pltpu.CompilerParams(use_tc_tiling_on_sc=False),
)
def _deg_call(dst_hbm, zeros_hbm, ones_hbm, out_hbm,
              idx_v, ones_v, buf_v, acc_sh, sem):
    c = lax.axis_index("c")
    s = lax.axis_index("s")
    wid = s * NC + c
    rows = pl.ds(s * RPT, RPT)

    pltpu.sync_copy(dst_hbm.at[wid], idx_v)
    pltpu.sync_copy(ones_hbm, ones_v)
    pltpu.sync_copy(zeros_hbm.at[rows], buf_v)
    pltpu.sync_copy(buf_v, acc_sh.at[rows])
    plsc.subcore_barrier()

    def body(j, carry):
        pltpu.sync_copy(ones_v, acc_sh.at[idx_v.at[j]], add=True)
        return carry

    lax.fori_loop(0, NCH, body, 0)
    plsc.subcore_barrier()
    pltpu.sync_copy(acc_sh.at[rows], buf_v)
    pltpu.sync_copy(buf_v, out_hbm.at[c].at[rows])


@functools.partial(
    pl.kernel,
    mesh=_MESH,
    out_type=jax.ShapeDtypeStruct((NC, NPAD, D), jnp.float32),
    scratch_types=[
        pltpu.VMEM((NCH, CH), jnp.int32),
        pltpu.VMEM((NCH, CH), jnp.int32),
        [pltpu.VMEM((CH, D), jnp.float32)] * NB,
        pltpu.VMEM((RPT, D), jnp.float32),
        pltpu.VMEM_SHARED((NPAD, D), jnp.float32),
        [pltpu.SemaphoreType.DMA] * NB,
        [pltpu.SemaphoreType.DMA] * NB,
    ],
    compiler_params=pltpu.CompilerParams(use_tc_tiling_on_sc=False),
)
def _edge_call(src_hbm, dst_hbm, mh_hbm, zeros_hbm, out_hbm,
               si_v, di_v, rb, buf_v, acc_sh, gsem, ssem):
    c = lax.axis_index("c")
    s = lax.axis_index("s")
    wid = s * NC + c
    rows = pl.ds(s * RPT, RPT)

    pltpu.sync_copy(src_hbm.at[wid], si_v)
    pltpu.sync_copy(dst_hbm.at[wid], di_v)
    pltpu.sync_copy(zeros_hbm.at[rows], buf_v)
    pltpu.sync_copy(buf_v, acc_sh.at[rows])
    plsc.subcore_barrier()

    for b in range(NB):
        pltpu.async_copy(mh_hbm.at[si_v.at[b]], rb[b], gsem[b])

    def body(i, carry):
        base = i * NB
        for b in range(NB):
            j = base + b
            pltpu.make_async_copy(mh_hbm.at[si_v.at[j]], rb[b],
                                  gsem[b]).wait()
            pltpu.async_copy(rb[b], acc_sh.at[di_v.at[j]], ssem[b],
                             add=True)
        for b in range(NB):
            nxt = base + NB + b
            pltpu.make_async_copy(rb[b], acc_sh.at[di_v.at[base + b]],
                                  ssem[b]).wait()

            @pl.when(nxt < NCH)
            def _():
                pltpu.async_copy(mh_hbm.at[si_v.at[nxt]], rb[b], gsem[b])

        return carry

    lax.fori_loop(0, NCH // NB, body, 0)
    for j in range(NCH - NCH % NB, NCH):
        b = j % NB
        pltpu.make_async_copy(mh_hbm.at[si_v.at[j]], rb[b], gsem[b]).wait()
        pltpu.sync_copy(rb[b], acc_sh.at[di_v.at[j]], add=True)
    plsc.subcore_barrier()
    pltpu.sync_copy(acc_sh.at[rows], buf_v)
    pltpu.sync_copy(buf_v, out_hbm.at[c].at[rows])


_BR = 1000
_GRID = N // _BR


def _tc_front_body(x_ref, wi_ref, bi_ref, w0_ref, d0_ref, d1_ref,
                   mh_ref, dis_ref):
    pre = jnp.dot(x_ref[...], wi_ref[...],
                  preferred_element_type=jnp.float32) + bi_ref[...]
    h = jnp.where(pre >= 0, pre, 0.01 * pre)
    deg = 1.0 + d0_ref[...] + d1_ref[...]
    dis = lax.rsqrt(deg)
    m = jnp.dot(h, w0_ref[...], preferred_element_type=jnp.float32)
    mh_ref[...] = dis * m
    dis_ref[...] = dis


def _tc_front(x, w_in, b_in, w0, d0, d1):
    return pl.pallas_call(
        _tc_front_body,
        grid=(_GRID,),
        in_specs=[
            pl.BlockSpec((_BR, F_IN), lambda i: (i, 0)),
            pl.BlockSpec((F_IN, D), lambda i: (0, 0)),
            pl.BlockSpec((1, D), lambda i: (0, 0)),
            pl.BlockSpec((D, D), lambda i: (0, 0)),
            pl.BlockSpec((_BR, 1), lambda i: (i, 0)),
            pl.BlockSpec((_BR, 1), lambda i: (i, 0)),
        ],
        out_specs=[
            pl.BlockSpec((_BR, D), lambda i: (i, 0)),
            pl.BlockSpec((_BR, 1), lambda i: (i, 0)),
        ],
        out_shape=[
            jax.ShapeDtypeStruct((N, D), jnp.float32),
            jax.ShapeDtypeStruct((N, 1), jnp.float32),
        ],
    )(x, w_in, b_in, w0, d0, d1)


def _tc_layer_body(a_ref, mh_ref, dis_ref, b_ref, w_ref, h_ref, mhn_ref):
    dis = dis_ref[...]
    pre = (a_ref[0] + a_ref[1] + mh_ref[...]) * dis + b_ref[...]
    h = jnp.where(pre >= 0, pre, 0.01 * pre)
    h_ref[...] = h
    m = jnp.dot(h, w_ref[...], preferred_element_type=jnp.float32)
    mhn_ref[...] = dis * m


def _tc_layer(accp, mh, dis, b, w):
    return pl.pallas_call(
        _tc_layer_body,
        grid=(_GRID,),
        in_specs=[
            pl.BlockSpec((NC, _BR, D), lambda i: (0, i, 0)),
            pl.BlockSpec((_BR, D), lambda i: (i, 0)),
            pl.BlockSpec((_BR, 1), lambda i: (i, 0)),
            pl.BlockSpec((1, D), lambda i: (0, 0)),
            pl.BlockSpec((D, D), lambda i: (0, 0)),
        ],
        out_specs=[
            pl.BlockSpec((_BR, D), lambda i: (i, 0)),
            pl.BlockSpec((_BR, D), lambda i: (i, 0)),
        ],
        out_shape=[
            jax.ShapeDtypeStruct((N, D), jnp.float32),
            jax.ShapeDtypeStruct((N, D), jnp.float32),
        ],
    )(accp, mh, dis, b, w)


def kernel(x, edge_index, W_in, b_in, Wc, bc):
    src = edge_index[0].reshape(NW, NCH, CH)
    dst = edge_index[1].reshape(NW, NCH, CH)
    zeros_d = jnp.zeros((NPAD, D), jnp.float32)
    zeros_16 = jnp.zeros((NPAD, 16), jnp.float32)
    ones_16 = jnp.ones((CH, 16), jnp.float32)

    degp = _deg_call(dst, zeros_16, ones_16)
    d0 = degp[0, :N, 0:1]
    d1 = degp[1, :N, 0:1]
    mh, dis = _tc_front(x, W_in, b_in.reshape(1, D), Wc[0], d0, d1)

    hs = []
    for k in range(L):
        accp = _edge_call(src, dst, mh, zeros_d)[:, :N]
        h, mh = _tc_layer(accp, mh, dis, bc[k].reshape(1, D),
                          Wc[(k + 1) % L])
        hs.append(h)

    xs = jnp.stack([h.reshape(-1) for h in hs], axis=1)
    return (hs[-1], hs[-1], xs)


# speedup vs baseline: 29.2531x; 1.0566x over previous
import functools

import jax
import jax.numpy as jnp
from jax import lax
from jax.experimental import pallas as pl
from jax.experimental.pallas import tpu as pltpu
from jax.experimental.pallas import tpu_sc as plsc

N = 10000
E = 320000
F_IN = 128
D = 64
L = 4

NC = 2
NS = 16
NW = NC * NS
EPW = E // NW
CH = 125
NCH = EPW // CH
NPAD = 10240
NB = 3
RPT = NPAD // NS

_MESH = plsc.VectorSubcoreMesh(core_axis_name="c", subcore_axis_name="s")


@functools.partial(
    pl.kernel,
    mesh=_MESH,
    out_type=jax.ShapeDtypeStruct((NC, NPAD, 16), jnp.float32),
    scratch_types=[
        pltpu.VMEM((NCH, CH), jnp.int32),
        pltpu.VMEM((CH, 16), jnp.float32),
        pltpu.VMEM((RPT, 16), jnp.float32),
        pltpu.VMEM_SHARED((NPAD, 16), jnp.float32),
        pltpu.SemaphoreType.DMA,
    ],
    compiler_params=pltpu.CompilerParams(use_tc_tiling_on_sc=False),
)
def _deg_call(dst_hbm, zeros_hbm, ones_hbm, out_hbm,
              idx_v, ones_v, buf_v, acc_sh, sem):
    c = lax.axis_index("c")
    s = lax.axis_index("s")
    wid = s * NC + c
    rows = pl.ds(s * RPT, RPT)

    pltpu.sync_copy(dst_hbm.at[wid], idx_v)
    pltpu.sync_copy(ones_hbm, ones_v)
    pltpu.sync_copy(zeros_hbm.at[rows], buf_v)
    pltpu.sync_copy(buf_v, acc_sh.at[rows])
    plsc.subcore_barrier()

    def body(j, carry):
        pltpu.sync_copy(ones_v, acc_sh.at[idx_v.at[j]], add=True)
        return carry

    lax.fori_loop(0, NCH, body, 0)
    plsc.subcore_barrier()
    pltpu.sync_copy(acc_sh.at[rows], buf_v)
    pltpu.sync_copy(buf_v, out_hbm.at[c].at[rows])


@functools.partial(
    pl.kernel,
    mesh=_MESH,
    out_type=jax.ShapeDtypeStruct((NC, NPAD, D), jnp.float32),
    scratch_types=[
        pltpu.VMEM((NCH, CH), jnp.int32),
        pltpu.VMEM((NCH, CH), jnp.int32),
        [pltpu.VMEM((CH, D), jnp.float32)] * NB,
        pltpu.VMEM((RPT, D), jnp.float32),
        pltpu.VMEM_SHARED((NPAD, D), jnp.float32),
        [pltpu.SemaphoreType.DMA] * NB,
        [pltpu.SemaphoreType.DMA] * NB,
    ],
    compiler_params=pltpu.CompilerParams(use_tc_tiling_on_sc=False),
)
def _edge_call(src_hbm, dst_hbm, mh_hbm, zeros_hbm, out_hbm,
               si_v, di_v, rb, buf_v, acc_sh, gsem, ssem):
    c = lax.axis_index("c")
    s = lax.axis_index("s")
    wid = s * NC + c
    rows = pl.ds(s * RPT, RPT)

    pltpu.sync_copy(src_hbm.at[wid], si_v)
    pltpu.sync_copy(dst_hbm.at[wid], di_v)
    pltpu.sync_copy(zeros_hbm.at[rows], buf_v)
    pltpu.sync_copy(buf_v, acc_sh.at[rows])
    plsc.subcore_barrier()

    for b in range(NB):
        pltpu.async_copy(mh_hbm.at[si_v.at[b]], rb[b], gsem[b])

    def body(i, carry):
        base = i * NB
        for b in range(NB):
            j = base + b
            pltpu.make_async_copy(mh_hbm.at[si_v.at[j]], rb[b],
                                  gsem[b]).wait()
            pltpu.async_copy(rb[b], acc_sh.at[di_v.at[j]], ssem[b],
                             add=True)
        for b in range(NB):
            nxt = base + NB + b
            pltpu.make_async_copy(rb[b], acc_sh.at[di_v.at[base + b]],
                                  ssem[b]).wait()

            @pl.when(nxt < NCH)
            def _():
                pltpu.async_copy(mh_hbm.at[si_v.at[nxt]], rb[b], gsem[b])

        return carry

    lax.fori_loop(0, NCH // NB, body, 0)
    for j in range(NCH - NCH % NB, NCH):
        b = j % NB
        pltpu.make_async_copy(mh_hbm.at[si_v.at[j]], rb[b], gsem[b]).wait()
        pltpu.sync_copy(rb[b], acc_sh.at[di_v.at[j]], add=True)
    plsc.subcore_barrier()
    pltpu.sync_copy(acc_sh.at[rows], buf_v)
    pltpu.sync_copy(buf_v, out_hbm.at[c].at[rows])


_BR = 1000
_GRID = N // _BR


def _tc_front_body(x_ref, wi_ref, bi_ref, w0_ref, d0_ref, d1_ref,
                   mh_ref, dis_ref):
    pre = jnp.dot(x_ref[...], wi_ref[...],
                  preferred_element_type=jnp.float32) + bi_ref[...]
    h = jnp.where(pre >= 0, pre, 0.01 * pre)
    deg = 1.0 + d0_ref[...] + d1_ref[...]
    dis = lax.rsqrt(deg)
    m = jnp.dot(h, w0_ref[...], preferred_element_type=jnp.float32)
    mh_ref[...] = dis * m
    dis_ref[...] = dis


def _tc_front(x, w_in, b_in, w0, d0, d1):
    return pl.pallas_call(
        _tc_front_body,
        grid=(_GRID,),
        in_specs=[
            pl.BlockSpec((_BR, F_IN), lambda i: (i, 0)),
            pl.BlockSpec((F_IN, D), lambda i: (0, 0)),
            pl.BlockSpec((1, D), lambda i: (0, 0)),
            pl.BlockSpec((D, D), lambda i: (0, 0)),
            pl.BlockSpec((_BR, 1), lambda i: (i, 0)),
            pl.BlockSpec((_BR, 1), lambda i: (i, 0)),
        ],
        out_specs=[
            pl.BlockSpec((_BR, D), lambda i: (i, 0)),
            pl.BlockSpec((_BR, 1), lambda i: (i, 0)),
        ],
        out_shape=[
            jax.ShapeDtypeStruct((N, D), jnp.float32),
            jax.ShapeDtypeStruct((N, 1), jnp.float32),
        ],
    )(x, w_in, b_in, w0, d0, d1)


def _tc_layer_body(a_ref, mh_ref, dis_ref, b_ref, w_ref, h_ref, mhn_ref):
    dis = dis_ref[...]
    pre = (a_ref[0] + a_ref[1] + mh_ref[...]) * dis + b_ref[...]
    h = jnp.where(pre >= 0, pre, 0.01 * pre)
    h_ref[...] = h
    m = jnp.dot(h, w_ref[...], preferred_element_type=jnp.float32)
    mhn_ref[...] = dis * m


def _tc_layer(accp, mh, dis, b, w):
    return pl.pallas_call(
        _tc_layer_body,
        grid=(_GRID,),
        in_specs=[
            pl.BlockSpec((NC, _BR, D), lambda i: (0, i, 0)),
            pl.BlockSpec((_BR, D), lambda i: (i, 0)),
            pl.BlockSpec((_BR, 1), lambda i: (i, 0)),
            pl.BlockSpec((1, D), lambda i: (0, 0)),
            pl.BlockSpec((D, D), lambda i: (0, 0)),
        ],
        out_specs=[
            pl.BlockSpec((_BR, D), lambda i: (i, 0)),
            pl.BlockSpec((_BR, D), lambda i: (i, 0)),
        ],
        out_shape=[
            jax.ShapeDtypeStruct((N, D), jnp.float32),
            jax.ShapeDtypeStruct((N, D), jnp.float32),
        ],
    )(accp, mh, dis, b, w)


def kernel(x, edge_index, W_in, b_in, Wc, bc):
    src = edge_index[0].reshape(NW, NCH, CH)
    dst = edge_index[1].reshape(NW, NCH, CH)
    zeros_d = jnp.zeros((NPAD, D), jnp.float32)
    zeros_16 = jnp.zeros((NPAD, 16), jnp.float32)
    ones_16 = jnp.ones((CH, 16), jnp.float32)

    degp = _deg_call(dst, zeros_16, ones_16)
    d0 = degp[0, :N, 0:1]
    d1 = degp[1, :N, 0:1]
    mh, dis = _tc_front(x, W_in, b_in.reshape(1, D), Wc[0], d0, d1)

    hs = []
    for k in range(L):
        accp = _edge_call(src, dst, mh, zeros_d)
        h, mh = _tc_layer(accp, mh, dis, bc[k].reshape(1, D),
                          Wc[(k + 1) % L])
        hs.append(h)

    xs = jnp.stack([h.reshape(-1) for h in hs], axis=1)
    return (hs[-1], hs[-1], xs)
